# Initial kernel scaffold; baseline (speedup 1.0000x reference)
import functools

import jax
import jax.numpy as jnp
from jax import lax
from jax.experimental import pallas as pl
from jax.experimental.pallas import tpu as pltpu
from jax.experimental.pallas import tpu_sc as plsc


_LANE = 128
_K = 8


@functools.partial(jax.jit, static_argnums=(2, 3))
def _sc_gather(idx2d, weight, rows_per_worker, num_workers):
    n_rows, lane = idx2d.shape
    d = weight.shape[1]
    b_per_w = rows_per_worker * lane
    chunk = _K * lane
    n_rounds = rows_per_worker // _K

    mesh = plsc.VectorSubcoreMesh(core_axis_name="c", subcore_axis_name="s")

    @functools.partial(
        pl.kernel,
        out_type=jax.ShapeDtypeStruct((n_rows * lane, d), jnp.float32),
        mesh=mesh,
        scratch_types=[
            pltpu.VMEM((rows_per_worker, lane), jnp.int32),
            pltpu.VMEM((chunk, d), jnp.float32),
            pltpu.SemaphoreType.DMA,
        ],
    )
    def k(weight_hbm, idx_hbm, out_hbm, idx_v, rows_v, sem):
        num_cores = mesh.num_cores
        wid = lax.axis_index("s") * num_cores + lax.axis_index("c")
        row0 = wid * rows_per_worker
        out0 = wid * b_per_w
        pltpu.sync_copy(idx_hbm.at[pl.ds(row0, rows_per_worker)], idx_v)

        @pl.loop(0, n_rounds)
        def _round(g):
            copies = []
            for j in range(_K):
                copies.append(
                    pltpu.async_copy(
                        weight_hbm.at[idx_v.at[g * _K + j]],
                        rows_v.at[pl.ds(j * lane, lane)],
                        sem,
                    )
                )
            for c in copies:
                c.wait()
            pltpu.sync_copy(rows_v, out_hbm.at[pl.ds(out0 + g * chunk, chunk)])

    return k(weight, idx2d)


def kernel(token_ids, weight):
    b, s = token_ids.shape
    v, d = weight.shape
    total = b * s
    idx = token_ids.reshape(total).astype(jnp.int32)

    num_workers = 32
    assert total % (num_workers * _LANE * _K) == 0
    rows_per_worker = total // (num_workers * _LANE)
    idx2d = idx.reshape(total // _LANE, _LANE)

    out = _sc_gather(idx2d, weight, rows_per_worker, num_workers)
    return out.reshape(b, s, d)


# speedup vs baseline: 1.1027x; 1.1027x over previous
import functools

import jax
import jax.numpy as jnp
from jax import lax
from jax.experimental import pallas as pl
from jax.experimental.pallas import tpu as pltpu
from jax.experimental.pallas import tpu_sc as plsc


_LANE = 128
_K = 8


@functools.partial(jax.jit, static_argnums=(2, 3))
def _sc_gather(idx2d, weight, rows_per_worker, num_workers):
    n_rows, lane = idx2d.shape
    d = weight.shape[1]
    b_per_w = rows_per_worker * lane
    chunk = _K * lane
    n_rounds = rows_per_worker // _K

    mesh = plsc.VectorSubcoreMesh(core_axis_name="c", subcore_axis_name="s")

    @functools.partial(
        pl.kernel,
        out_type=jax.ShapeDtypeStruct((n_rows * lane, d), jnp.float32),
        mesh=mesh,
        scratch_types=[
            pltpu.VMEM((rows_per_worker, lane), jnp.int32),
            pltpu.VMEM((chunk, d), jnp.float32),
            pltpu.SemaphoreType.DMA,
        ],
        compiler_params=pltpu.CompilerParams(use_tc_tiling_on_sc=False),
    )
    def k(weight_hbm, idx_hbm, out_hbm, idx_v, rows_v, sem):
        num_cores = mesh.num_cores
        wid = lax.axis_index("s") * num_cores + lax.axis_index("c")
        row0 = wid * rows_per_worker
        out0 = wid * b_per_w
        pltpu.sync_copy(idx_hbm.at[pl.ds(row0, rows_per_worker)], idx_v)

        @pl.loop(0, n_rounds)
        def _round(g):
            copies = []
            for j in range(_K):
                copies.append(
                    pltpu.async_copy(
                        weight_hbm.at[idx_v.at[g * _K + j]],
                        rows_v.at[pl.ds(j * lane, lane)],
                        sem,
                    )
                )
            for c in copies:
                c.wait()
            pltpu.sync_copy(rows_v, out_hbm.at[pl.ds(out0 + g * chunk, chunk)])

    return k(weight, idx2d)


def kernel(token_ids, weight):
    b, s = token_ids.shape
    v, d = weight.shape
    total = b * s
    idx = token_ids.reshape(total).astype(jnp.int32)

    num_workers = 32
    assert total % (num_workers * _LANE * _K) == 0
    rows_per_worker = total // (num_workers * _LANE)
    idx2d = idx.reshape(total // _LANE, _LANE)

    out = _sc_gather(idx2d, weight, rows_per_worker, num_workers)
    return out.reshape(b, s, d)


# speedup vs baseline: 1.1123x; 1.0088x over previous
import functools

import jax
import jax.numpy as jnp
from jax import lax
from jax.experimental import pallas as pl
from jax.experimental.pallas import tpu as pltpu
from jax.experimental.pallas import tpu_sc as plsc


_LANE = 128
_K = 10


@functools.partial(jax.jit, static_argnums=(2, 3))
def _sc_gather(idx2d, weight, rows_per_worker, num_workers):
    n_rows, lane = idx2d.shape
    d = weight.shape[1]
    b_per_w = rows_per_worker * lane
    chunk = _K * lane
    n_rounds = rows_per_worker // _K
    half = n_rounds // 2

    mesh = plsc.VectorSubcoreMesh(core_axis_name="c", subcore_axis_name="s")

    @functools.partial(
        pl.kernel,
        out_type=jax.ShapeDtypeStruct((n_rows * lane, d), jnp.float32),
        mesh=mesh,
        scratch_types=[
            pltpu.VMEM((rows_per_worker, lane), jnp.int32),
            pltpu.VMEM((chunk, d), jnp.float32),
            pltpu.VMEM((chunk, d), jnp.float32),
            pltpu.SemaphoreType.DMA,
            pltpu.SemaphoreType.DMA,
            pltpu.SemaphoreType.DMA,
            pltpu.SemaphoreType.DMA,
        ],
        compiler_params=pltpu.CompilerParams(use_tc_tiling_on_sc=False),
    )
    def k(weight_hbm, idx_hbm, out_hbm, idx_v, buf0, buf1, gs0, gs1, os0, os1):
        num_cores = mesh.num_cores
        wid = lax.axis_index("s") * num_cores + lax.axis_index("c")
        row0 = wid * rows_per_worker
        out0 = wid * b_per_w
        pltpu.sync_copy(idx_hbm.at[pl.ds(row0, rows_per_worker)], idx_v)

        def fire(g, buf, sem):
            for j in range(_K):
                pltpu.async_copy(
                    weight_hbm.at[idx_v.at[g * _K + j]],
                    buf.at[pl.ds(j * lane, lane)],
                    sem,
                )

        def drain_gather(buf, sem):
            pltpu.make_async_copy(
                out_hbm.at[pl.ds(out0, chunk)], buf, sem).wait()

        def start_out(g, buf, sem):
            pltpu.async_copy(buf, out_hbm.at[pl.ds(out0 + g * chunk, chunk)],
                             sem)

        def wait_out(g, buf, sem):
            pltpu.make_async_copy(
                buf, out_hbm.at[pl.ds(out0 + g * chunk, chunk)], sem).wait()

        fire(0, buf0, gs0)

        @pl.loop(0, half)
        def _h(h):
            a = 2 * h
            b = a + 1

            @pl.when(h > 0)
            def _():
                wait_out(b, buf1, os1)

            fire(b, buf1, gs1)
            drain_gather(buf0, gs0)
            start_out(a, buf0, os0)
            drain_gather(buf1, gs1)
            wait_out(a, buf0, os0)

            @pl.when(h + 1 < half)
            def _():
                fire(a + 2, buf0, gs0)

            start_out(b, buf1, os1)

        wait_out(n_rounds - 1, buf1, os1)

    return k(weight, idx2d)


def kernel(token_ids, weight):
    b, s = token_ids.shape
    v, d = weight.shape
    total = b * s
    idx = token_ids.reshape(total).astype(jnp.int32)

    num_workers = 32
    rows_per_worker = total // (num_workers * _LANE)
    assert total == num_workers * rows_per_worker * _LANE
    assert rows_per_worker % (2 * _K) == 0
    idx2d = idx.reshape(total // _LANE, _LANE)

    out = _sc_gather(idx2d, weight, rows_per_worker, num_workers)
    return out.reshape(b, s, d)


# speedup vs baseline: 1.8047x; 1.6225x over previous
import functools

import jax
import jax.numpy as jnp
from jax import lax
from jax.experimental import pallas as pl
from jax.experimental.pallas import tpu as pltpu
from jax.experimental.pallas import tpu_sc as plsc


_NB = 16
_NW = 32


@functools.partial(jax.jit, static_argnums=())
def _sc_gather(token_ids, weight):
    bsz, seq = token_ids.shape
    d = weight.shape[1]
    b_per_w = bsz // _NW
    n_rounds = b_per_w // _NB
    half = n_rounds // 2

    mesh = plsc.VectorSubcoreMesh(core_axis_name="c", subcore_axis_name="s")

    @functools.partial(
        pl.kernel,
        out_type=jax.ShapeDtypeStruct((bsz, seq, d), jnp.float32),
        mesh=mesh,
        scratch_types=[
            pltpu.VMEM((b_per_w, seq), jnp.int32),
            pltpu.VMEM((_NB, seq, d), jnp.float32),
            pltpu.VMEM((_NB, seq, d), jnp.float32),
            pltpu.SemaphoreType.DMA,
            pltpu.SemaphoreType.DMA,
            pltpu.SemaphoreType.DMA,
            pltpu.SemaphoreType.DMA,
        ],
        compiler_params=pltpu.CompilerParams(use_tc_tiling_on_sc=False),
    )
    def k(weight_hbm, idx_hbm, out_hbm, idx_v, buf0, buf1, gs0, gs1, os0, os1):
        num_cores = mesh.num_cores
        wid = lax.axis_index("s") * num_cores + lax.axis_index("c")
        bat0 = wid * b_per_w
        pltpu.sync_copy(idx_hbm.at[pl.ds(bat0, b_per_w)], idx_v)

        def out_slice(g):
            return out_hbm.at[pl.ds(bat0 + g * _NB, _NB)]

        def fire(g, buf, sem):
            for j in range(_NB):
                pltpu.async_copy(
                    weight_hbm.at[idx_v.at[g * _NB + j]],
                    buf.at[j],
                    sem,
                )

        def drain_gather(g, buf, sem):
            pltpu.make_async_copy(out_slice(g), buf, sem).wait()

        def start_out(g, buf, sem):
            pltpu.async_copy(buf, out_slice(g), sem)

        def wait_out(g, buf, sem):
            pltpu.make_async_copy(buf, out_slice(g), sem).wait()

        fire(0, buf0, gs0)

        @pl.loop(0, half)
        def _h(h):
            a = 2 * h
            b = a + 1

            @pl.when(h > 0)
            def _():
                wait_out(b, buf1, os1)

            fire(b, buf1, gs1)
            drain_gather(a, buf0, gs0)
            start_out(a, buf0, os0)
            drain_gather(b, buf1, gs1)
            wait_out(a, buf0, os0)

            @pl.when(h + 1 < half)
            def _():
                fire(a + 2, buf0, gs0)

            start_out(b, buf1, os1)

        wait_out(n_rounds - 1, buf1, os1)

    return k(weight, token_ids)


def kernel(token_ids, weight):
    bsz, seq = token_ids.shape
    idx = token_ids.astype(jnp.int32)
    assert bsz % (_NW * _NB * 2) == 0
    return _sc_gather(idx, weight)


# speedup vs baseline: 2.5357x; 1.4050x over previous
import functools

import jax
import jax.numpy as jnp
from jax import lax
from jax.experimental import pallas as pl
from jax.experimental.pallas import tpu as pltpu
from jax.experimental.pallas import tpu_sc as plsc


_NB = 16
_NW = 32
_SEQ_PAD = 56


@jax.jit
def _sc_gather(token_ids, weight):
    bsz, seq = token_ids.shape
    d = weight.shape[1]
    b_per_w = bsz // _NW
    n_rounds = b_per_w // _NB
    half = n_rounds // 2

    mesh = plsc.VectorSubcoreMesh(core_axis_name="c", subcore_axis_name="s")

    @functools.partial(
        pl.kernel,
        out_type=jax.ShapeDtypeStruct((bsz * _SEQ_PAD, 128), jnp.float32),
        mesh=mesh,
        scratch_types=[
            pltpu.VMEM((b_per_w, seq), jnp.int32),
            pltpu.VMEM((_NB, seq, d), jnp.float32),
            pltpu.VMEM((_NB, seq, d), jnp.float32),
            pltpu.SemaphoreType.DMA,
            pltpu.SemaphoreType.DMA,
            pltpu.SemaphoreType.DMA,
            pltpu.SemaphoreType.DMA,
        ],
        compiler_params=pltpu.CompilerParams(use_tc_tiling_on_sc=False),
    )
    def k(weight_hbm, idx_hbm, out_hbm, idx_v, buf0, buf1, gs0, gs1, os0, os1):
        num_cores = mesh.num_cores
        wid = lax.axis_index("s") * num_cores + lax.axis_index("c")
        bat0 = wid * b_per_w
        pltpu.sync_copy(idx_hbm.at[pl.ds(bat0, b_per_w)], idx_v)

        def fire(g, buf, sem):
            for j in range(_NB):
                pltpu.async_copy(
                    weight_hbm.at[idx_v.at[g * _NB + j]],
                    buf.at[j],
                    sem,
                )

        def drain_gather(buf, sem):
            for j in range(_NB):
                pltpu.make_async_copy(
                    weight_hbm.at[pl.ds(0, seq)], buf.at[j], sem).wait()

        def out_copies(g, buf, sem):
            for j in range(_NB):
                row0 = (bat0 + g * _NB + j) * _SEQ_PAD
                pltpu.async_copy(
                    buf.at[j],
                    out_hbm.at[pl.ds(row0, seq), pl.ds(0, d)],
                    sem,
                )

        def wait_out(buf, sem):
            for j in range(_NB):
                pltpu.make_async_copy(
                    buf.at[j],
                    out_hbm.at[pl.ds(0, seq), pl.ds(0, d)],
                    sem).wait()

        fire(0, buf0, gs0)

        @pl.loop(0, half)
        def _h(h):
            a = 2 * h
            b = a + 1

            @pl.when(h > 0)
            def _():
                wait_out(buf1, os1)

            fire(b, buf1, gs1)
            drain_gather(buf0, gs0)
            out_copies(a, buf0, os0)
            drain_gather(buf1, gs1)
            wait_out(buf0, os0)

            @pl.when(h + 1 < half)
            def _():
                fire(a + 2, buf0, gs0)

            out_copies(b, buf1, os1)

        wait_out(buf1, os1)

    return k(weight, token_ids)


def kernel(token_ids, weight):
    bsz, seq = token_ids.shape
    d = weight.shape[1]
    idx = token_ids.astype(jnp.int32)
    assert bsz % (_NW * _NB * 2) == 0 and seq <= _SEQ_PAD
    out = _sc_gather(idx, weight)
    return out.reshape(bsz, _SEQ_PAD, 128)[:, :seq, :d]
